# baseline (speedup 1.0000x reference)
import functools

import jax
import jax.numpy as jnp
from jax import lax
from jax.experimental import pallas as pl
from jax.experimental.pallas import tpu as pltpu
from jax.experimental.pallas import tpu_sc as plsc

_NUM_CLASSES = 1000000
_FEAT = 64
_BATCH = 16384
_LAMBDA = 0.003

_INFO = plsc.get_sparse_core_info()
_NC, _NS, _L = _INFO.num_cores, _INFO.num_subcores, _INFO.num_lanes
_NW = _NC * _NS
_BPW = _BATCH // _NW
_NG = _BPW // _L
_NBUF = 4
_AHEAD = 3
_FVEC = _FEAT // _L
_EROW = _BPW * _FEAT // 128

_TCOLS = _NUM_CLASSES // 128
_TPW = _TCOLS // _NW
_TEXTRA = _TCOLS - _TPW * _NW
_TAIL = _TCOLS * 128
_STR = 137

_COMPILER = pltpu.CompilerParams(
    use_tc_tiling_on_sc=True, needs_layout_passes=False)


def _transpose_body(centersT_hbm, out_hbm, in_v, out_v, sem_i, sem_o):
    wid = lax.axis_index("s") * _NC + lax.axis_index("c")
    base = wid * _TPW
    iota = lax.iota(jnp.int32, _L)

    def in_descr(i, tc):
        s = lax.rem(i, 2)
        return pltpu.make_async_copy(
            centersT_hbm.at[:, pl.ds(tc * 128, 128)],
            in_v.at[s, :, pl.ds(0, 128)], sem_i)

    def out_descr(i, tc):
        s = lax.rem(i, 2)
        return pltpu.make_async_copy(
            out_v.at[s], out_hbm.at[pl.ds(tc * 128, 128), :], sem_o)

    def compute(i):
        s = lax.rem(i, 2)
        sv = jnp.full((_L,), s, jnp.int32)

        def row(r, carry):
            rv = jnp.full((_L,), r, jnp.int32)
            for c in range(_FVEC):
                v = plsc.load_gather(in_v, [sv, iota + (c * _L), rv])
                out_v[s, r, pl.ds(c * _L, _L)] = v
            return carry

        lax.fori_loop(0, 128, row, 0)

    in_descr(0, base).start()

    def step(i, carry):
        @pl.when(i < _TPW - 1)
        def _():
            in_descr(i + 1, base + i + 1).start()
        in_descr(i, base + i).wait()

        @pl.when(i >= 2)
        def _():
            out_descr(i - 2, base + i - 2).wait()
        compute(i)
        out_descr(i, base + i).start()
        return carry

    lax.fori_loop(0, _TPW, step, 0)
    out_descr(_TPW - 2, base + _TPW - 2).wait()
    out_descr(_TPW - 1, base + _TPW - 1).wait()

    @pl.when(wid < _TEXTRA)
    def _():
        tc = _NW * _TPW + wid
        in_descr(0, tc).start()
        in_descr(0, tc).wait()
        compute(0)
        out_descr(0, tc).start()
        out_descr(0, tc).wait()


def _loss_body(labels_hbm, emb_hbm, table_hbm, tail_hbm, out_hbm,
               lab_v, blocks_v, emb_v, tail_v, out_v, sem_b, sem_e):
    wid = lax.axis_index("s") * _NC + lax.axis_index("c")
    base = wid * _BPW

    pltpu.sync_copy(labels_hbm.at[pl.ds(base, _BPW)], lab_v)
    pltpu.sync_copy(tail_hbm, tail_v)
    emb_cp = pltpu.async_copy(
        emb_hbm.at[pl.ds(wid * _EROW, _EROW), :], emb_v, sem_e)

    def fire(g):
        lv = lab_v[pl.ds(g * _L, _L)]
        av = lax.shift_left(lax.shift_right_logical(lv, 3), 3)
        s = lax.rem(g, _NBUF)
        for b in range(_L):
            al = pl.multiple_of(av[b], 8)
            pltpu.async_copy(
                table_hbm.at[pl.ds(al, 8), :], blocks_v.at[s, b], sem_b)

    for g in range(_AHEAD):
        fire(g)
    emb_cp.wait()

    def step(g, accs):
        @pl.when(g < _NG - _AHEAD)
        def _():
            fire(g + _AHEAD)
        s = lax.rem(g, _NBUF)
        for b in range(_L):
            pltpu.make_async_copy(
                table_hbm.at[pl.ds(0, 8), :],
                blocks_v.at[s, b], sem_b).wait()
        lv = lab_v[pl.ds(g * _L, _L)]
        out = list(accs)
        for b in range(_L):
            lb = lv[b]
            ob = lax.bitwise_and(lb, 7)
            in_tail = lb >= _TAIL
            tl = lax.max(lb - _TAIL, 0)
            k = g * (_L // 2) + (b // 2)
            for c in range(_FVEC):
                ev = emb_v[k, pl.ds((b % 2) * _FEAT + c * _L, _L)]
                cv = blocks_v[s, b, ob, pl.ds(c * _L, _L)]
                tv = tail_v[tl, pl.ds(c * _L, _L)]
                d = ev - jnp.where(in_tail, tv, cv)
                out[c] = out[c] + d * d
        return tuple(out)

    zero = jnp.zeros((_L,), jnp.float32)
    accs = lax.fori_loop(0, _NG, step, (zero,) * _FVEC)
    out_v[...] = (accs[0] + accs[1]) + (accs[2] + accs[3])
    pltpu.sync_copy(out_v, out_hbm.at[pl.ds(wid * _L, _L)])


@jax.jit
def _center_loss_partials(labels, emb128, centersT, tail):
    mesh = plsc.VectorSubcoreMesh(core_axis_name="c", subcore_axis_name="s")
    table = functools.partial(
        pl.kernel,
        mesh=mesh,
        out_type=jax.ShapeDtypeStruct((_NUM_CLASSES, _FEAT), jnp.float32),
        scratch_types=[
            pltpu.VMEM((2, _FEAT, _STR), jnp.float32),
            pltpu.VMEM((2, 128, _FEAT), jnp.float32),
            pltpu.SemaphoreType.DMA,
            pltpu.SemaphoreType.DMA,
        ],
        compiler_params=_COMPILER,
    )(_transpose_body)(centersT)
    partials = functools.partial(
        pl.kernel,
        mesh=mesh,
        out_type=jax.ShapeDtypeStruct((_NW * _L,), jnp.float32),
        scratch_types=[
            pltpu.VMEM((_BPW,), jnp.int32),
            pltpu.VMEM((_NBUF, _L, 8, _FEAT), jnp.float32),
            pltpu.VMEM((_EROW, 128), jnp.float32),
            pltpu.VMEM((_FEAT, _FEAT), jnp.float32),
            pltpu.VMEM((_L,), jnp.float32),
            pltpu.SemaphoreType.DMA,
            pltpu.SemaphoreType.DMA,
        ],
        compiler_params=_COMPILER,
    )(_loss_body)(labels, emb128, table, tail)
    return partials


def kernel(embeddings, labels, centers):
    emb128 = embeddings.reshape(_BATCH * _FEAT // 128, 128)
    tail = centers[_TAIL:]
    partials = _center_loss_partials(
        labels.astype(jnp.int32), emb128, centers.T, tail)
    return jnp.sum(partials) * (_LAMBDA / (_BATCH * _FEAT))
